# Initial kernel scaffold; baseline (speedup 1.0000x reference)
import jax
import jax.numpy as jnp
from jax.experimental import pallas as pl


def kernel(input_x, instr_x, lora_A_param, W_B, W_P):
    raise NotImplementedError("write your pallas kernel here")


# speedup vs baseline: 1.5884x; 1.5884x over previous
import functools

import jax
import jax.numpy as jnp
from jax.experimental import pallas as pl
from jax.experimental.pallas import tpu as pltpu

_DIM = 4096
_E = 64
_R = 8
_IN = 2 * _DIM
_BZ = 4
_SEQ = 2048
_SCHUNK = 128
_NSTEPS = _SEQ // _SCHUNK
_WPCHUNK = _IN


def _softmax_lanes(x):
    m = jnp.max(x, axis=-1, keepdims=True)
    e = jnp.exp(x - m)
    return e / jnp.sum(e, axis=-1, keepdims=True)


def _dot_t(a, b):
    return jax.lax.dot_general(
        a, b, (((1,), (1,)), ((), ())), preferred_element_type=jnp.float32)


def _body(x_ref, wp_ref, wb_ref, instr_ref, la_ref, out_ref, acc_ref, wps_ref):
    s = pl.program_id(0)

    @pl.when(s == 0)
    def _():
        acc_ref[...] = jnp.zeros_like(acc_ref)

    blk = x_ref[...]
    for b in range(_BZ):
        acc_ref[b : b + 1, :] += jnp.sum(blk[b], axis=0, keepdims=True)

    @pl.when(s < _R)
    def _():
        ones = jnp.ones((1, _WPCHUNK), jnp.float32)
        row = _dot_t(ones, wp_ref[...])
        wps_ref[pl.ds(s, 1), :] = row

    @pl.when(s == _NSTEPS - 1)
    def _():
        mean = acc_ref[...] * (1.0 / _SEQ)
        wb = wb_ref[...]
        l1 = _dot_t(instr_ref[...], wb[:, :_DIM]) + _dot_t(mean, wb[:, _DIM:])
        s1 = _softmax_lanes(l1)

        s2 = jnp.zeros((_BZ, _E), jnp.float32)
        for r in range(_R):
            a_r = jnp.sum(la_ref[:, r, :], axis=-1, keepdims=True)
            l2r = a_r * wps_ref[r : r + 1, :]
            s2 = s2 + _softmax_lanes(l2r)
        logits = s1 + _softmax_lanes(s2)

        iota = jax.lax.broadcasted_iota(jnp.int32, (_BZ, _E), 1)
        iota8 = jax.lax.broadcasted_iota(jnp.int32, (_BZ, _R), 1)
        vals = logits
        out = jnp.zeros((_BZ, _R), jnp.int32)
        for k in range(_R):
            m = jnp.max(vals, axis=-1, keepdims=True)
            idx = jnp.min(jnp.where(vals >= m, iota, _E), axis=-1,
                          keepdims=True)
            out = jnp.where(iota8 == k, idx, out)
            vals = jnp.where(iota == idx, -jnp.inf, vals)
        out_ref[...] = out


@jax.jit
def kernel(input_x, instr_x, lora_A_param, W_B, W_P):
    return pl.pallas_call(
        _body,
        grid=(_NSTEPS,),
        in_specs=[
            pl.BlockSpec((_BZ, _SCHUNK, _DIM), lambda s: (0, s, 0)),
            pl.BlockSpec((_E, _WPCHUNK), lambda s: (0, jnp.minimum(s, _R - 1))),
            pl.BlockSpec((_E, _IN), lambda s: (0, 0)),
            pl.BlockSpec((_BZ, _DIM), lambda s: (0, 0)),
            pl.BlockSpec((_BZ, _R, _IN), lambda s: (0, 0, 0)),
        ],
        out_specs=pl.BlockSpec((_BZ, _R), lambda s: (0, 0)),
        out_shape=jax.ShapeDtypeStruct((_BZ, _R), jnp.int32),
        scratch_shapes=[
            pltpu.VMEM((_BZ, _DIM), jnp.float32),
            pltpu.VMEM((_R, _E), jnp.float32),
        ],
    )(input_x, W_P, W_B, instr_x, lora_A_param)
